# baseline (speedup 1.0000x reference)
import functools

import jax
import jax.numpy as jnp
import numpy as np
from jax import lax
from jax.experimental import pallas as pl
from jax.experimental.pallas import tpu as pltpu
from jax.experimental.pallas import tpu_sc as plsc

PERM = np.array([(c // 2) * 32 + 2 * r + (c % 2)
                 for c in range(4) for r in range(16)])
INV_PERM = np.argsort(PERM)

D = 64
L = 200
LP = 208
M = 4
B = 1024
NCHUNK = 8
CW = 104
LW = 26
NBUF = 4
DC = 3 * D
NTAB = 3
NWORKERS = 32
NB = B // NWORKERS
FLAT = LP * M


def _body(story_hbm, posf_hbm, ct_hbm, out_hbm,
          story_f0, story_f1, pos_res, posb, m1, m2, m3,
          rows0, rows1, rows2, rows3,
          sbuf, outb, sem0, sem1, sem2, sem3, sem_s):
    wid = lax.axis_index("s") * 2 + lax.axis_index("c")
    iota = lax.iota(jnp.int32, 16)

    pltpu.sync_copy(posf_hbm, pos_res)

    rows = (rows0, rows1, rows2, rows3)
    sems = (sem0, sem1, sem2, sem3)
    ms = (m1, m2, m3)
    z4 = tuple(jnp.zeros((16,), jnp.float32) for _ in range(4))

    def issue(sf, j):
        return pltpu.async_copy(
            ct_hbm.at[sf.at[pl.ds(j * CW, CW)]],
            rows[j % NBUF], sems[j % NBUF])

    pltpu.sync_copy(story_hbm.at[wid * NB], story_f0)
    for j in range(3):
        issue(story_f0, j)

    def half_body(i, sf, sfn):
        b = wid * NB + i
        bn = jnp.minimum(b + 1, B - 1)
        s_desc = pltpu.async_copy(story_hbm.at[bn], sfn, sem_s)

        def pos_body(q, carry):
            v = plsc.load_gather(sf, [(q * 16 + iota) * M])
            npad = jnp.where(v != 0, 1, 0).astype(jnp.int32)
            cs = plsc.cumsum(npad) + carry
            posb[pl.ds(q * 16, 16)] = jnp.where(v != 0, cs, 0) * D
            return jnp.max(cs)

        pl.loop(0, LP // 16, init_carry=jnp.int32(0))(pos_body)

        for j in range(NCHUNK):
            if j + 3 < NCHUNK:
                issue(sf, j + 3)
            elif j + 3 == NCHUNK:
                s_desc.wait()
                issue(sfn, 0)
            else:
                issue(sfn, j + 3 - NCHUNK)
            pltpu.make_async_copy(
                ct_hbm.at[sf.at[pl.ds(j * CW, CW)]],
                rows[j % NBUF], sems[j % NBUF]).wait()
            rb = rows[j % NBUF]

            def acc_body(t, rb=rb, j=j):
                sl0 = j * LW + t
                pb = posb[pl.ds(sl0, 16)][0]
                es = [pos_res[pl.ds(pb + blk * 16, 16)] for blk in range(4)]
                for h in range(NTAB):
                    for half in range(2):
                        sl32 = pl.ds(h * D + half * 32, 32)
                        ups = [plsc.unpack(rb[4 * t + i, sl32],
                                           format=plsc.PackFormat.INTERLEAVED)
                               for i in range(4)]
                        for part in range(2):
                            blk = 2 * half + part
                            a = ups[0][part] + ups[1][part]
                            a2 = ups[2][part] + ups[3][part]
                            ms[h][sl0, pl.ds(blk * 16, 16)] = (a + a2) + es[blk]

            pl.loop(0, LW)(acc_body)

        def red_body(sl0, acc):
            return tuple(acc[c] + m1[sl0, pl.ds(c * 16, 16)] for c in range(4))

        u = pl.loop(0, L, init_carry=z4, unroll=2)(red_body)
        u = tuple(uc * (1.0 / L) for uc in u)

        for hop in range(2):
            mh_a = (m1, m2)[hop]
            mh_c = (m2, m3)[hop]

            def s_chunk(q, mh_a=mh_a, u=u):
                lv = q * 16 + iota
                acc = jnp.zeros((16,), jnp.float32)
                for c in range(4):
                    for r in range(16):
                        g = plsc.load_gather(
                            mh_a, [lv, jnp.full((16,), c * 16 + r, jnp.int32)])
                        acc = acc + g * u[c][r]
                s = jnp.where((q < 12) | (iota < 8), acc, -1e30)
                sbuf[pl.ds(q * 16, 16)] = s

            pl.loop(0, LP // 16)(s_chunk)

            def mx_body(q, acc):
                return jnp.maximum(acc, sbuf[pl.ds(q * 16, 16)])

            vm = pl.loop(0, LP // 16,
                         init_carry=jnp.full((16,), -1e30, jnp.float32))(mx_body)
            mx = jnp.max(vm)

            def exp_body(q, acc):
                e = jnp.exp(sbuf[pl.ds(q * 16, 16)] - mx)
                sbuf[pl.ds(q * 16, 16)] = e
                return acc + e

            sv = pl.loop(0, LP // 16,
                         init_carry=jnp.zeros((16,), jnp.float32))(exp_body)
            rinv = 1.0 / jnp.full((16,), jnp.sum(sv), jnp.float32)

            def o_body(sl0, acc, mh_c=mh_c):
                pv = sbuf[pl.ds(sl0, 16)][0]
                return tuple(acc[c] + mh_c[sl0, pl.ds(c * 16, 16)] * pv
                             for c in range(4))

            o = pl.loop(0, L, init_carry=z4, unroll=2)(o_body)
            u = tuple(u[c] + o[c] * rinv for c in range(4))

        for c in range(4):
            plsc.store_scatter(
                outb, [jnp.full((16,), i, jnp.int32),
                       (c // 2) * 32 + (c % 2) + 2 * iota],
                u[c])

    def pair_body(p):
        half_body(2 * p, story_f0, story_f1)
        half_body(2 * p + 1, story_f1, story_f0)

    pl.loop(0, NB // 2)(pair_body)

    for k in range(3):
        pltpu.make_async_copy(
            ct_hbm.at[story_f0.at[pl.ds(k * CW, CW)]],
            rows[k], sems[k]).wait()

    pltpu.sync_copy(outb, out_hbm.at[pl.ds(wid * NB, NB)])


@functools.partial(jax.jit, static_argnames=())
def _mem_nn(story_p, posf, ctab):
    mesh = plsc.VectorSubcoreMesh(
        core_axis_name="c", subcore_axis_name="s", num_cores=2,
        num_subcores=16)
    call = pl.kernel(
        _body,
        out_type=jax.ShapeDtypeStruct((B, D), jnp.float32),
        mesh=mesh,
        scratch_types=[
            pltpu.VMEM((FLAT,), jnp.int32),
            pltpu.VMEM((FLAT,), jnp.int32),
            pltpu.VMEM((201 * D,), jnp.float32),
            pltpu.VMEM((LP + 16,), jnp.int32),
            pltpu.VMEM((LP, D), jnp.float32),
            pltpu.VMEM((LP, D), jnp.float32),
            pltpu.VMEM((LP, D), jnp.float32),
            pltpu.VMEM((CW, DC), jnp.bfloat16),
            pltpu.VMEM((CW, DC), jnp.bfloat16),
            pltpu.VMEM((CW, DC), jnp.bfloat16),
            pltpu.VMEM((CW, DC), jnp.bfloat16),
            pltpu.VMEM((LP + 16,), jnp.float32),
            pltpu.VMEM((NB, D), jnp.float32),
            pltpu.SemaphoreType.DMA,
            pltpu.SemaphoreType.DMA,
            pltpu.SemaphoreType.DMA,
            pltpu.SemaphoreType.DMA,
            pltpu.SemaphoreType.DMA,
        ],
        compiler_params=pltpu.CompilerParams(
            needs_layout_passes=False, use_tc_tiling_on_sc=False),
        name="mem_nn_sc",
    )
    return call(story_p, posf, ctab)


def kernel(story, pos_emb, C_0, C_1, C_2, C_3):
    del C_0
    l, b, m = story.shape
    st = jnp.transpose(story, (1, 0, 2)).reshape(b, l * m)
    st = jnp.pad(st, ((0, 0), (0, FLAT - l * m)))
    posf = pos_emb[:, PERM].reshape(-1)
    ctab = jnp.concatenate(
        [C_1, C_2, C_3], axis=1).astype(jnp.bfloat16)
    return _mem_nn(st, posf, ctab)


# speedup vs baseline: 1.0117x; 1.0117x over previous
import functools

import jax
import jax.numpy as jnp
import numpy as np
from jax import lax
from jax.experimental import pallas as pl
from jax.experimental.pallas import tpu as pltpu
from jax.experimental.pallas import tpu_sc as plsc

PERM = np.array([(c // 2) * 32 + 2 * r + (c % 2)
                 for c in range(4) for r in range(16)])
INV_PERM = np.argsort(PERM)

D = 64
L = 200
LP = 208
M = 4
B = 1024
NCHUNK = 8
CW = 104
LW = 26
NBUF = 4
DC = 3 * D
NTAB = 3
NWORKERS = 32
NB = B // NWORKERS
FLAT = LP * M


def _body(story_hbm, posf_hbm, ct_hbm, out_hbm,
          story_f0, story_f1, pos_res, posb, m1, m2, m3,
          rows0, rows1, rows2, rows3,
          sbuf, outb, sem0, sem1, sem2, sem3, sem_s):
    wid = lax.axis_index("s") * 2 + lax.axis_index("c")
    iota = lax.iota(jnp.int32, 16)

    pltpu.sync_copy(posf_hbm, pos_res)

    rows = (rows0, rows1, rows2, rows3)
    sems = (sem0, sem1, sem2, sem3)
    ms = (m1, m2, m3)
    z4 = tuple(jnp.zeros((16,), jnp.float32) for _ in range(4))

    def issue(sf, j):
        return pltpu.async_copy(
            ct_hbm.at[sf.at[pl.ds(j * CW, CW)]],
            rows[j % NBUF], sems[j % NBUF])

    pltpu.sync_copy(story_hbm.at[wid * NB], story_f0)
    for j in range(3):
        issue(story_f0, j)

    def half_body(i, sf, sfn):
        b = wid * NB + i
        bn = jnp.minimum(b + 1, B - 1)
        s_desc = pltpu.async_copy(story_hbm.at[bn], sfn, sem_s)

        def pos_body(q, carry):
            v = plsc.load_gather(sf, [(q * 16 + iota) * M])
            npad = jnp.where(v != 0, 1, 0).astype(jnp.int32)
            cs = plsc.cumsum(npad) + carry
            posb[pl.ds(q * 16, 16)] = jnp.where(v != 0, cs, 0) * D
            return jnp.max(cs)

        pl.loop(0, LP // 16, init_carry=jnp.int32(0))(pos_body)
        u1acc = z4

        for j in range(NCHUNK):
            if j + 3 < NCHUNK:
                issue(sf, j + 3)
            elif j + 3 == NCHUNK:
                s_desc.wait()
                issue(sfn, 0)
            else:
                issue(sfn, j + 3 - NCHUNK)
            pltpu.make_async_copy(
                ct_hbm.at[sf.at[pl.ds(j * CW, CW)]],
                rows[j % NBUF], sems[j % NBUF]).wait()
            rb = rows[j % NBUF]

            def acc_body(t, acc, rb=rb, j=j):
                sl0 = j * LW + t
                pb = posb[pl.ds(sl0, 16)][0]
                es = [pos_res[pl.ds(pb + blk * 16, 16)] for blk in range(4)]
                acc_out = list(acc)
                for h in range(NTAB):
                    for half in range(2):
                        sl32 = pl.ds(h * D + half * 32, 32)
                        ups = [plsc.unpack(rb[4 * t + i, sl32],
                                           format=plsc.PackFormat.INTERLEAVED)
                               for i in range(4)]
                        for part in range(2):
                            blk = 2 * half + part
                            a = ups[0][part] + ups[1][part]
                            a2 = ups[2][part] + ups[3][part]
                            val = (a + a2) + es[blk]
                            ms[h][sl0, pl.ds(blk * 16, 16)] = val
                            if h == 0:
                                acc_out[blk] = acc_out[blk] + val
                return tuple(acc_out)

            u1acc = pl.loop(0, LW, init_carry=u1acc)(acc_body)

        u = tuple(uc * (1.0 / L) for uc in u1acc)

        for hop in range(2):
            mh_a = (m1, m2)[hop]
            mh_c = (m2, m3)[hop]

            def s_chunk(q, mh_a=mh_a, u=u):
                lv = q * 16 + iota
                acc = jnp.zeros((16,), jnp.float32)
                for c in range(4):
                    for r in range(16):
                        g = plsc.load_gather(
                            mh_a, [lv, jnp.full((16,), c * 16 + r, jnp.int32)])
                        acc = acc + g * u[c][r]
                s = jnp.where((q < 12) | (iota < 8), acc, -1e30)
                sbuf[pl.ds(q * 16, 16)] = s

            pl.loop(0, LP // 16)(s_chunk)

            def mx_body(q, acc):
                return jnp.maximum(acc, sbuf[pl.ds(q * 16, 16)])

            vm = pl.loop(0, LP // 16,
                         init_carry=jnp.full((16,), -1e30, jnp.float32))(mx_body)
            mx = jnp.max(vm)

            def exp_body(q, acc):
                e = jnp.exp(sbuf[pl.ds(q * 16, 16)] - mx)
                sbuf[pl.ds(q * 16, 16)] = e
                return acc + e

            sv = pl.loop(0, LP // 16,
                         init_carry=jnp.zeros((16,), jnp.float32))(exp_body)
            rinv = 1.0 / jnp.full((16,), jnp.sum(sv), jnp.float32)

            def o_body(sl0, acc, mh_c=mh_c):
                pv = sbuf[pl.ds(sl0, 16)][0]
                return tuple(acc[c] + mh_c[sl0, pl.ds(c * 16, 16)] * pv
                             for c in range(4))

            o = pl.loop(0, L, init_carry=z4, unroll=2)(o_body)
            u = tuple(u[c] + o[c] * rinv for c in range(4))

        for c in range(4):
            outb[i, pl.ds(c * 16, 16)] = u[c]

    def pair_body(p):
        half_body(2 * p, story_f0, story_f1)
        half_body(2 * p + 1, story_f1, story_f0)

    pl.loop(0, NB // 2)(pair_body)

    for k in range(3):
        pltpu.make_async_copy(
            ct_hbm.at[story_f0.at[pl.ds(k * CW, CW)]],
            rows[k], sems[k]).wait()

    pltpu.sync_copy(outb, out_hbm.at[pl.ds(wid * NB, NB)])


@functools.partial(jax.jit, static_argnames=())
def _mem_nn(story_p, posf, ctab):
    mesh = plsc.VectorSubcoreMesh(
        core_axis_name="c", subcore_axis_name="s", num_cores=2,
        num_subcores=16)
    call = pl.kernel(
        _body,
        out_type=jax.ShapeDtypeStruct((B, D), jnp.float32),
        mesh=mesh,
        scratch_types=[
            pltpu.VMEM((FLAT,), jnp.int32),
            pltpu.VMEM((FLAT,), jnp.int32),
            pltpu.VMEM((201 * D,), jnp.float32),
            pltpu.VMEM((LP + 16,), jnp.int32),
            pltpu.VMEM((LP, D), jnp.float32),
            pltpu.VMEM((LP, D), jnp.float32),
            pltpu.VMEM((LP, D), jnp.float32),
            pltpu.VMEM((CW, DC), jnp.bfloat16),
            pltpu.VMEM((CW, DC), jnp.bfloat16),
            pltpu.VMEM((CW, DC), jnp.bfloat16),
            pltpu.VMEM((CW, DC), jnp.bfloat16),
            pltpu.VMEM((LP + 16,), jnp.float32),
            pltpu.VMEM((NB, D), jnp.float32),
            pltpu.SemaphoreType.DMA,
            pltpu.SemaphoreType.DMA,
            pltpu.SemaphoreType.DMA,
            pltpu.SemaphoreType.DMA,
            pltpu.SemaphoreType.DMA,
        ],
        compiler_params=pltpu.CompilerParams(
            needs_layout_passes=False, use_tc_tiling_on_sc=False),
        name="mem_nn_sc",
    )
    return call(story_p, posf, ctab)


def kernel(story, pos_emb, C_0, C_1, C_2, C_3):
    del C_0
    l, b, m = story.shape
    st = jnp.transpose(story, (1, 0, 2)).reshape(b, l * m)
    st = jnp.pad(st, ((0, 0), (0, FLAT - l * m)))
    posf = pos_emb[:, PERM].reshape(-1)
    ctab = jnp.concatenate(
        [C_1, C_2, C_3], axis=1).astype(jnp.bfloat16)
    out = _mem_nn(st, posf, ctab)
    return out[:, INV_PERM]


# speedup vs baseline: 1.0170x; 1.0052x over previous
import functools

import jax
import jax.numpy as jnp
import numpy as np
from jax import lax
from jax.experimental import pallas as pl
from jax.experimental.pallas import tpu as pltpu
from jax.experimental.pallas import tpu_sc as plsc

PERM = np.array([(c // 2) * 32 + 2 * r + (c % 2)
                 for c in range(4) for r in range(16)])
INV_PERM = np.argsort(PERM)

D = 64
L = 200
LP = 208
M = 4
B = 1024
NCHUNK = 4
CW = 208
LW = 52
NBUF = 3
LOOK = NBUF - 1
DC = 3 * D
NTAB = 3
NWORKERS = 32
NB = B // NWORKERS
FLAT = LP * M


def _body(story_hbm, posf_hbm, ct_hbm, out_hbm,
          story_f0, story_f1, pos_res, posb, m1, m2, m3,
          rows0, rows1, rows2,
          sbuf, outb, sem0, sem1, sem2, sem_s):
    wid = lax.axis_index("s") * 2 + lax.axis_index("c")
    iota = lax.iota(jnp.int32, 16)

    pltpu.sync_copy(posf_hbm, pos_res)

    rows = (rows0, rows1, rows2)
    sems = (sem0, sem1, sem2)
    ms = (m1, m2, m3)
    z4 = tuple(jnp.zeros((16,), jnp.float32) for _ in range(4))

    def issue(sf, j):
        return pltpu.async_copy(
            ct_hbm.at[sf.at[pl.ds(j * CW, CW)]],
            rows[j % NBUF], sems[j % NBUF])

    pltpu.sync_copy(story_hbm.at[wid * NB], story_f0)
    for j in range(LOOK):
        issue(story_f0, j)

    def half_body(i, sf, sfn):
        b = wid * NB + i
        bn = jnp.minimum(b + 1, B - 1)
        s_desc = pltpu.async_copy(story_hbm.at[bn], sfn, sem_s)

        def pos_body(q, carry):
            v = plsc.load_gather(sf, [(q * 16 + iota) * M])
            npad = jnp.where(v != 0, 1, 0).astype(jnp.int32)
            cs = plsc.cumsum(npad) + carry
            posb[pl.ds(q * 16, 16)] = jnp.where(v != 0, cs, 0) * D
            return jnp.max(cs)

        pl.loop(0, LP // 16, init_carry=jnp.int32(0))(pos_body)
        u1acc = z4

        for j in range(NCHUNK):
            if j + LOOK < NCHUNK:
                issue(sf, j + LOOK)
            elif j + LOOK == NCHUNK:
                s_desc.wait()
                issue(sfn, 0)
            else:
                issue(sfn, j + LOOK - NCHUNK)
            pltpu.make_async_copy(
                ct_hbm.at[sf.at[pl.ds(j * CW, CW)]],
                rows[j % NBUF], sems[j % NBUF]).wait()
            rb = rows[j % NBUF]

            def acc_body(t, acc, rb=rb, j=j):
                sl0 = j * LW + t
                pb = posb[pl.ds(sl0, 16)][0]
                es = [pos_res[pl.ds(pb + blk * 16, 16)] for blk in range(4)]
                acc_out = list(acc)
                for h in range(NTAB):
                    for half in range(2):
                        sl32 = pl.ds(h * D + half * 32, 32)
                        ups = [plsc.unpack(rb[4 * t + i, sl32],
                                           format=plsc.PackFormat.INTERLEAVED)
                               for i in range(4)]
                        for part in range(2):
                            blk = 2 * half + part
                            a = ups[0][part] + ups[1][part]
                            a2 = ups[2][part] + ups[3][part]
                            val = (a + a2) + es[blk]
                            ms[h][sl0, pl.ds(blk * 16, 16)] = val
                            if h == 0:
                                acc_out[blk] = acc_out[blk] + val
                return tuple(acc_out)

            u1acc = pl.loop(0, LW, init_carry=u1acc)(acc_body)

        u = tuple(uc * (1.0 / L) for uc in u1acc)

        for hop in range(2):
            mh_a = (m1, m2)[hop]
            mh_c = (m2, m3)[hop]

            def s_chunk(q, mh_a=mh_a, u=u):
                lv = q * 16 + iota
                acc = jnp.zeros((16,), jnp.float32)
                for c in range(4):
                    for r in range(16):
                        g = plsc.load_gather(
                            mh_a, [lv, jnp.full((16,), c * 16 + r, jnp.int32)])
                        acc = acc + g * u[c][r]
                s = jnp.where((q < 12) | (iota < 8), acc, -1e30)
                sbuf[pl.ds(q * 16, 16)] = s

            pl.loop(0, LP // 16)(s_chunk)

            def mx_body(q, acc):
                return jnp.maximum(acc, sbuf[pl.ds(q * 16, 16)])

            vm = pl.loop(0, LP // 16,
                         init_carry=jnp.full((16,), -1e30, jnp.float32))(mx_body)
            mx = jnp.max(vm)

            def exp_body(q, acc):
                e = jnp.exp(sbuf[pl.ds(q * 16, 16)] - mx)
                sbuf[pl.ds(q * 16, 16)] = e
                return acc + e

            sv = pl.loop(0, LP // 16,
                         init_carry=jnp.zeros((16,), jnp.float32))(exp_body)
            rinv = 1.0 / jnp.full((16,), jnp.sum(sv), jnp.float32)

            def o_body(sl0, acc, mh_c=mh_c):
                pv = sbuf[pl.ds(sl0, 16)][0]
                return tuple(acc[c] + mh_c[sl0, pl.ds(c * 16, 16)] * pv
                             for c in range(4))

            o = pl.loop(0, L, init_carry=z4, unroll=2)(o_body)
            u = tuple(u[c] + o[c] * rinv for c in range(4))

        for c in range(4):
            outb[i, pl.ds(c * 16, 16)] = u[c]

    def pair_body(p):
        half_body(2 * p, story_f0, story_f1)
        half_body(2 * p + 1, story_f1, story_f0)

    pl.loop(0, NB // 2)(pair_body)

    for k in range(LOOK):
        pltpu.make_async_copy(
            ct_hbm.at[story_f0.at[pl.ds(k * CW, CW)]],
            rows[k], sems[k]).wait()

    pltpu.sync_copy(outb, out_hbm.at[pl.ds(wid * NB, NB)])


@functools.partial(jax.jit, static_argnames=())
def _mem_nn(story_p, posf, ctab):
    mesh = plsc.VectorSubcoreMesh(
        core_axis_name="c", subcore_axis_name="s", num_cores=2,
        num_subcores=16)
    call = pl.kernel(
        _body,
        out_type=jax.ShapeDtypeStruct((B, D), jnp.float32),
        mesh=mesh,
        scratch_types=[
            pltpu.VMEM((FLAT,), jnp.int32),
            pltpu.VMEM((FLAT,), jnp.int32),
            pltpu.VMEM((201 * D,), jnp.float32),
            pltpu.VMEM((LP + 16,), jnp.int32),
            pltpu.VMEM((LP, D), jnp.float32),
            pltpu.VMEM((LP, D), jnp.float32),
            pltpu.VMEM((LP, D), jnp.float32),
            pltpu.VMEM((CW, DC), jnp.bfloat16),
            pltpu.VMEM((CW, DC), jnp.bfloat16),
            pltpu.VMEM((CW, DC), jnp.bfloat16),
            pltpu.VMEM((LP + 16,), jnp.float32),
            pltpu.VMEM((NB, D), jnp.float32),
            pltpu.SemaphoreType.DMA,
            pltpu.SemaphoreType.DMA,
            pltpu.SemaphoreType.DMA,
            pltpu.SemaphoreType.DMA,
        ],
        compiler_params=pltpu.CompilerParams(
            needs_layout_passes=False, use_tc_tiling_on_sc=False),
        name="mem_nn_sc",
    )
    return call(story_p, posf, ctab)


def kernel(story, pos_emb, C_0, C_1, C_2, C_3):
    del C_0
    l, b, m = story.shape
    st = jnp.transpose(story, (1, 0, 2)).reshape(b, l * m)
    st = jnp.pad(st, ((0, 0), (0, FLAT - l * m)))
    posf = pos_emb[:, PERM].reshape(-1)
    ctab = jnp.concatenate(
        [C_1, C_2, C_3], axis=1).astype(jnp.bfloat16)
    out = _mem_nn(st, posf, ctab)
    return out[:, INV_PERM]
